# Initial kernel scaffold; baseline (speedup 1.0000x reference)
import jax
import jax.numpy as jnp
from jax.experimental import pallas as pl


def kernel(input_val, n0_W1, n0_b1, n0_W2, n0_b2, n1_preW, n1_preb, n1_W1, n1_b1, n1_W2, n1_b2):
    raise NotImplementedError("write your pallas kernel here")


# speedup vs baseline: 1.1795x; 1.1795x over previous
import functools

import jax
import jax.numpy as jnp
from jax import lax
from jax.experimental import pallas as pl
from jax.experimental.pallas import tpu as pltpu
from jax.experimental.pallas import tpu_sc as plsc

_B = 4096
_E = 8
_BLK = 128
_P = _B + _E * _BLK
_NBLK = _P // _BLK


@functools.cache
def _make_sc_gather(V, D, N):
    info = plsc.get_sparse_core_info()
    NC, NS = info.num_cores, info.num_subcores
    NW = NC * NS
    assert N % NW == 0
    rows_pw = N // NW
    cap = min(128, rows_pw, (320 * 1024) // (D * 4))
    chunk = 8
    for c in range(8, cap + 1, 8):
        if rows_pw % c == 0:
            chunk = c
    nch = rows_pw // chunk
    mesh = plsc.VectorSubcoreMesh(core_axis_name="c", subcore_axis_name="s")

    @functools.partial(
        pl.kernel,
        out_type=jax.ShapeDtypeStruct((N, D), jnp.float32),
        mesh=mesh,
        scratch_types=[
            pltpu.VMEM((chunk,), jnp.int32),
            pltpu.VMEM((chunk, D), jnp.float32),
            pltpu.SemaphoreType.DMA,
        ],
    )
    def gather_kernel(table_hbm, idx_hbm, out_hbm, idx_v, rows_v, sem):
        wid = lax.axis_index("s") * NC + lax.axis_index("c")
        base = wid * rows_pw
        for c in range(nch):
            off = base + c * chunk
            pltpu.sync_copy(idx_hbm.at[pl.ds(off, chunk)], idx_v)
            pltpu.async_copy(table_hbm.at[idx_v], rows_v, sem).wait()
            pltpu.sync_copy(rows_v, out_hbm.at[pl.ds(off, chunk)])

    return gather_kernel


def _mlp0_body(be_ref, act_ref, x_ref, w1_ref, b1_ref, w2_ref, b2_ref, o_ref):
    i = pl.program_id(0)

    @pl.when(act_ref[i] != 0)
    def _():
        h = jnp.dot(x_ref[...], w1_ref[0], preferred_element_type=jnp.float32)
        h = jnp.maximum(h + b1_ref[0], 0.0)
        y = jnp.dot(h, w2_ref[0], preferred_element_type=jnp.float32)
        o_ref[...] = jnp.maximum(y + b2_ref[0], 0.0)


def _mlp0_call(be, act, x_s, w1, b1, w2, b2):
    grid_spec = pltpu.PrefetchScalarGridSpec(
        num_scalar_prefetch=2,
        grid=(_NBLK,),
        in_specs=[
            pl.BlockSpec((_BLK, 128), lambda i, be, act: (i, 0)),
            pl.BlockSpec((1, 128, 1024), lambda i, be, act: (be[i], 0, 0)),
            pl.BlockSpec((1, 1, 1024), lambda i, be, act: (be[i], 0, 0)),
            pl.BlockSpec((1, 1024, 1024), lambda i, be, act: (be[i], 0, 0)),
            pl.BlockSpec((1, 1, 1024), lambda i, be, act: (be[i], 0, 0)),
        ],
        out_specs=pl.BlockSpec((_BLK, 1024), lambda i, be, act: (i, 0)),
    )
    return pl.pallas_call(
        _mlp0_body,
        grid_spec=grid_spec,
        out_shape=jax.ShapeDtypeStruct((_P, 1024), jnp.float32),
    )(be, act, x_s, w1, b1.reshape(_E, 1, 1024), w2, b2.reshape(_E, 1, 1024))


def _mlp1_body(be_ref, act_ref, x0_ref, f1_ref, pw_ref, pb_ref, w1a_ref,
               w1b_ref, b1_ref, w2_ref, b2_ref, o_ref):
    i = pl.program_id(0)

    @pl.when(act_ref[i] != 0)
    def _():
        pre = jnp.dot(f1_ref[...], pw_ref[0], preferred_element_type=jnp.float32)
        pre = jnp.maximum(pre + pb_ref[0], 0.0)
        h = jnp.dot(x0_ref[...], w1a_ref[0], preferred_element_type=jnp.float32)
        h = h + jnp.dot(pre, w1b_ref[0], preferred_element_type=jnp.float32)
        h = jnp.maximum(h + b1_ref[0], 0.0)
        y = jnp.dot(h, w2_ref[0], preferred_element_type=jnp.float32)
        o_ref[...] = y + b2_ref[0]


def _mlp1_call(be, act, x0_b, f1_s, pw, pb, w1, b1, w2, b2):
    w1a = w1[:, :1024, :]
    w1b = w1[:, 1024:, :]
    grid_spec = pltpu.PrefetchScalarGridSpec(
        num_scalar_prefetch=2,
        grid=(_NBLK,),
        in_specs=[
            pl.BlockSpec((_BLK, 1024), lambda i, be, act: (i, 0)),
            pl.BlockSpec((_BLK, 128), lambda i, be, act: (i, 0)),
            pl.BlockSpec((1, 128, 512), lambda i, be, act: (be[i], 0, 0)),
            pl.BlockSpec((1, 1, 512), lambda i, be, act: (be[i], 0, 0)),
            pl.BlockSpec((1, 1024, 1024), lambda i, be, act: (be[i], 0, 0)),
            pl.BlockSpec((1, 512, 1024), lambda i, be, act: (be[i], 0, 0)),
            pl.BlockSpec((1, 1, 1024), lambda i, be, act: (be[i], 0, 0)),
            pl.BlockSpec((1, 1024, 512), lambda i, be, act: (be[i], 0, 0)),
            pl.BlockSpec((1, 1, 512), lambda i, be, act: (be[i], 0, 0)),
        ],
        out_specs=pl.BlockSpec((_BLK, 512), lambda i, be, act: (i, 0)),
    )
    return pl.pallas_call(
        _mlp1_body,
        grid_spec=grid_spec,
        out_shape=jax.ShapeDtypeStruct((_P, 512), jnp.float32),
    )(be, act, x0_b, f1_s, pw, pb.reshape(_E, 1, 512), w1a, w1b,
      b1.reshape(_E, 1, 1024), w2, b2.reshape(_E, 1, 512))


def _routing(oh):
    counts = jnp.sum(oh, axis=0).astype(jnp.int32)
    rank_all = jnp.cumsum(oh, axis=0) - oh
    idx = jnp.argmax(oh, axis=1).astype(jnp.int32)
    rank = jnp.take_along_axis(rank_all, idx[:, None], axis=1)[:, 0]
    rank = rank.astype(jnp.int32)
    pc = ((counts + _BLK - 1) // _BLK) * _BLK
    starts = jnp.concatenate(
        [jnp.zeros((1,), jnp.int32), jnp.cumsum(pc)[:-1].astype(jnp.int32)])
    dest = starts[idx] + rank
    row_ids = jnp.zeros((_P,), jnp.int32).at[dest].set(
        jnp.arange(_B, dtype=jnp.int32))
    r = jnp.arange(_NBLK, dtype=jnp.int32) * _BLK
    total = jnp.sum(pc)
    be = jnp.clip(
        jnp.searchsorted(starts, r, side="right").astype(jnp.int32) - 1, 0, _E - 1)
    act = (r < total).astype(jnp.int32)
    last_e = jnp.maximum(
        jnp.max(jnp.where(counts > 0, jnp.arange(_E, dtype=jnp.int32), -1)), 0)
    be = jnp.where(act == 1, be, last_e)
    return dest, row_ids, be, act


def kernel(input_val, n0_W1, n0_b1, n0_W2, n0_b2, n1_preW, n1_preb,
           n1_W1, n1_b1, n1_W2, n1_b2):
    feats0 = input_val[:, 0:128]
    feats1 = input_val[:, 128:256]
    oh0 = input_val[:, 256:264]
    oh1 = input_val[:, 264:272]

    dest0, rid0, be0, act0 = _routing(oh0)
    dest1, rid1, be1, act1 = _routing(oh1)
    g3_src = dest0[rid1]

    f0_s = _make_sc_gather(_B, 128, _P)(feats0, rid0)
    f1_s = _make_sc_gather(_B, 128, _P)(feats1, rid1)
    x0_s = _mlp0_call(be0, act0, f0_s, n0_W1, n0_b1, n0_W2, n0_b2)
    x0_b = _make_sc_gather(_P, 1024, _P)(x0_s, g3_src)
    out_s = _mlp1_call(be1, act1, x0_b, f1_s, n1_preW, n1_preb,
                       n1_W1, n1_b1, n1_W2, n1_b2)
    out = _make_sc_gather(_P, 512, _B)(out_s, dest1)
    return out
